# Initial kernel scaffold; baseline (speedup 1.0000x reference)
import functools

import jax
import jax.numpy as jnp
from jax import lax
from jax.experimental import pallas as pl
from jax.experimental.pallas import tpu as pltpu
from jax.experimental.pallas import tpu_sc as plsc

_NC = 2
_NS = 16
_NW = _NC * _NS
_KSUB = 8
_G = 10
_CHUNK = _KSUB * 128
_ROWS_PER_W = _G * _KSUB


def _pad_edges(row, col, w, n_nodes):
    e = row.shape[0]
    total = _NW * _G * _CHUNK
    pad = total - e
    fill = jnp.arange(pad, dtype=jnp.int32) % n_nodes
    rowp = jnp.concatenate([row, fill]).reshape(total // 128, 128)
    colp = jnp.concatenate([col, fill]).reshape(total // 128, 128)
    wp = jnp.concatenate([w, jnp.zeros((pad,), w.dtype)]).reshape(
        total // 128, 128)
    return rowp, colp, wp


def _sc_degree(colp, wp, zinit):
    n = zinit.shape[0]
    mesh = plsc.VectorSubcoreMesh(core_axis_name="c", subcore_axis_name="s")

    @functools.partial(
        pl.kernel,
        out_type=jax.ShapeDtypeStruct((_NC, n), jnp.float32),
        mesh=mesh,
        scratch_types=[
            pltpu.VMEM((_KSUB, 128), jnp.int32),
            pltpu.VMEM((_KSUB, 128), jnp.float32),
            pltpu.VMEM_SHARED((n,), jnp.float32),
        ],
    )
    def deg_kernel(col_hbm, w_hbm, z_hbm, out_hbm, col_v, w_v, acc_sh):
        cid = lax.axis_index("c")
        sid = lax.axis_index("s")
        wid = cid * _NS + sid

        @pl.when(sid == 0)
        def _():
            pltpu.sync_copy(z_hbm, acc_sh)

        plsc.subcore_barrier()

        base_row = wid * _ROWS_PER_W

        @pl.loop(0, _G)
        def _(g):
            r0 = base_row + g * _KSUB
            pltpu.sync_copy(col_hbm.at[pl.ds(r0, _KSUB)], col_v)
            pltpu.sync_copy(w_hbm.at[pl.ds(r0, _KSUB)], w_v)
            for j in range(_KSUB):
                pltpu.sync_copy(w_v.at[j], acc_sh.at[col_v.at[j]], add=True)

        plsc.subcore_barrier()

        @pl.when(sid < _NS - 1)
        def _():
            pltpu.sync_copy(acc_sh.at[pl.ds(sid * 640, 640)],
                            out_hbm.at[cid, pl.ds(sid * 640, 640)])

        @pl.when(sid == _NS - 1)
        def _():
            pltpu.sync_copy(acc_sh.at[pl.ds(9600, 400)],
                            out_hbm.at[cid, pl.ds(9600, 400)])

    return deg_kernel(colp, wp, zinit)


def _sc_gather_scale_scatter(table, zinit, rowp, colp, wp):
    n, d = table.shape
    stripe = n // _NS
    mesh = plsc.VectorSubcoreMesh(core_axis_name="c", subcore_axis_name="s")

    @functools.partial(
        pl.kernel,
        out_type=jax.ShapeDtypeStruct((_NC, n, d), jnp.float32),
        mesh=mesh,
        scratch_types=[
            pltpu.VMEM((_KSUB, 128), jnp.int32),
            pltpu.VMEM((_KSUB, 128), jnp.int32),
            pltpu.VMEM((_KSUB, 128), jnp.float32),
            pltpu.VMEM((_CHUNK, d), jnp.float32),
            pltpu.SemaphoreType.DMA,
            pltpu.VMEM_SHARED((n, d), jnp.float32),
        ],
    )
    def seg_kernel(tab_hbm, z_hbm, row_hbm, col_hbm, w_hbm, out_hbm,
                   row_v, col_v, w_v, msg_v, gsem, acc_sh):
        cid = lax.axis_index("c")
        sid = lax.axis_index("s")
        wid = cid * _NS + sid

        @pl.when(cid == 0)
        def _():
            pltpu.sync_copy(tab_hbm.at[pl.ds(sid * stripe, stripe)],
                            acc_sh.at[pl.ds(sid * stripe, stripe)])

        @pl.when(cid != 0)
        def _():
            pltpu.sync_copy(z_hbm.at[pl.ds(sid * stripe, stripe)],
                            acc_sh.at[pl.ds(sid * stripe, stripe)])

        plsc.subcore_barrier()

        base_row = wid * _ROWS_PER_W

        @pl.loop(0, _G)
        def _(g):
            r0 = base_row + g * _KSUB
            pltpu.sync_copy(row_hbm.at[pl.ds(r0, _KSUB)], row_v)
            pltpu.sync_copy(col_hbm.at[pl.ds(r0, _KSUB)], col_v)
            pltpu.sync_copy(w_hbm.at[pl.ds(r0, _KSUB)], w_v)
            copies = []
            for j in range(_KSUB):
                copies.append(pltpu.async_copy(
                    tab_hbm.at[row_v.at[j]],
                    msg_v.at[pl.ds(j * 128, 128)], gsem))
            for c in copies:
                c.wait()

            @pl.loop(0, _KSUB)
            def _(j):
                jv = jnp.full((16,), j, jnp.int32)

                @pl.loop(0, 128)
                def _(i):
                    iv = jnp.full((16,), i, jnp.int32)
                    wv = plsc.load_gather(w_v, [jv, iv])
                    base = j * 128 + i
                    for t in range(d // 16):
                        sl = (base, pl.ds(t * 16, 16))
                        msg_v[sl] = msg_v[sl] * wv

            for j in range(_KSUB):
                pltpu.sync_copy(msg_v.at[pl.ds(j * 128, 128)],
                                acc_sh.at[col_v.at[j]], add=True)

        plsc.subcore_barrier()

        pltpu.sync_copy(acc_sh.at[pl.ds(sid * stripe, stripe)],
                        out_hbm.at[cid, pl.ds(sid * stripe, stripe)])

    return seg_kernel(table, zinit, rowp, colp, wp)


def _tc_matmul(x, w):
    n = x.shape[0]
    m = w.shape[1]

    def body(x_ref, w_ref, o_ref):
        o_ref[...] = jnp.dot(x_ref[...], w_ref[...],
                             preferred_element_type=jnp.float32)

    return pl.pallas_call(
        body, out_shape=jax.ShapeDtypeStruct((n, m), jnp.float32))(x, w)


def _tc_deg_scale(h, degt):
    n, m = h.shape

    def body(h_ref, d_ref, hp_ref, dis_ref):
        deg = d_ref[:, 0:1] + d_ref[:, 1:2] + 1.0
        dis = lax.rsqrt(deg)
        dis_ref[...] = dis
        hp_ref[...] = h_ref[...] * dis

    return pl.pallas_call(
        body,
        out_shape=[jax.ShapeDtypeStruct((n, m), jnp.float32),
                   jax.ShapeDtypeStruct((n, 1), jnp.float32)])(h, degt)


def _tc_layer1_finish(a0, a1, dis, b1, w2):
    n = a0.shape[0]
    m = w2.shape[1]

    def body(a0_ref, a1_ref, dis_ref, b_ref, w_ref, o_ref):
        z = jax.nn.relu((a0_ref[...] + a1_ref[...]) * dis_ref[...]
                        + b_ref[...])
        o_ref[...] = jnp.dot(z, w_ref[...],
                             preferred_element_type=jnp.float32) * dis_ref[...]

    return pl.pallas_call(
        body, out_shape=jax.ShapeDtypeStruct((n, m), jnp.float32))(
            a0, a1, dis, b1, w2)


def _tc_layer2_finish(a0, a1, dis, b2):
    n, m = a0.shape

    def body(a0_ref, a1_ref, dis_ref, b_ref, o_ref):
        o = (a0_ref[...] + a1_ref[...]) * dis_ref[...] + b_ref[...]
        mx = jnp.max(o, axis=1, keepdims=True)
        ex = jnp.exp(o - mx)
        lse = jnp.log(jnp.sum(ex, axis=1, keepdims=True)) + mx
        o_ref[...] = o - lse

    return pl.pallas_call(
        body, out_shape=jax.ShapeDtypeStruct((n, m), jnp.float32))(
            a0, a1, dis, b2)


def kernel(x, edge_index, edge_weight, W1, b1, W2, b2):
    n = x.shape[0]
    nhid = W1.shape[1]
    ncls = W2.shape[1]

    rowp, colp, wp = _pad_edges(edge_index[0], edge_index[1], edge_weight, n)
    z1d = jnp.zeros((n,), jnp.float32)
    z64 = jnp.zeros((n, nhid), jnp.float32)
    z16 = jnp.zeros((n, ncls), jnp.float32)

    degp = _sc_degree(colp, wp, z1d)
    h = _tc_matmul(x, W1)
    h1p, dis = _tc_deg_scale(h, degp.T)
    acc1 = _sc_gather_scale_scatter(h1p, z64, rowp, colp, wp)
    h2p = _tc_layer1_finish(acc1[0], acc1[1], dis,
                            b1.reshape(1, nhid), W2)
    acc2 = _sc_gather_scale_scatter(h2p, z16, rowp, colp, wp)
    return _tc_layer2_finish(acc2[0], acc2[1], dis,
                             b2.reshape(1, ncls))


# speedup vs baseline: 23.4441x; 23.4441x over previous
import dataclasses
import functools

import jax
import jax.numpy as jnp
from jax import lax
from jax.experimental import pallas as pl
from jax.experimental.pallas import tpu as pltpu
from jax.experimental.pallas import tpu_sc as plsc

_NC = 2
_NS = 16
_NW = _NC * _NS
_KSUB = 8
_G = 10
_CHUNK = _KSUB * 128
_ROWS_PER_W = _G * _KSUB


def _pad_edges(row, col, w, n_nodes):
    e = row.shape[0]
    total = _NW * _G * _CHUNK
    pad = total - e
    fill = jnp.arange(pad, dtype=jnp.int32) % n_nodes
    rowp = jnp.concatenate([row, fill]).reshape(total // 128, 128)
    colp = jnp.concatenate([col, fill]).reshape(total // 128, 128)
    wp = jnp.concatenate([w, jnp.zeros((pad,), w.dtype)]).reshape(
        total // 128, 128)
    return rowp, colp, wp


def _sc_degree(colp, wp, zinit):
    n = zinit.shape[0]
    mesh = plsc.VectorSubcoreMesh(core_axis_name="c", subcore_axis_name="s")

    @functools.partial(
        pl.kernel,
        out_type=jax.ShapeDtypeStruct((_NC, n), jnp.float32),
        mesh=mesh,
        scratch_types=[
            pltpu.VMEM((_KSUB, 128), jnp.int32),
            pltpu.VMEM((_KSUB, 128), jnp.float32),
            pltpu.VMEM_SHARED((n,), jnp.float32),
        ],
    )
    def deg_kernel(col_hbm, w_hbm, z_hbm, out_hbm, col_v, w_v, acc_sh):
        cid = lax.axis_index("c")
        sid = lax.axis_index("s")
        wid = cid * _NS + sid

        @pl.when(sid == 0)
        def _():
            pltpu.sync_copy(z_hbm, acc_sh)

        plsc.subcore_barrier()

        base_row = wid * _ROWS_PER_W

        @pl.loop(0, _G)
        def _(g):
            r0 = base_row + g * _KSUB
            pltpu.sync_copy(col_hbm.at[pl.ds(r0, _KSUB)], col_v)
            pltpu.sync_copy(w_hbm.at[pl.ds(r0, _KSUB)], w_v)
            for j in range(_KSUB):
                pltpu.sync_copy(w_v.at[j], acc_sh.at[col_v.at[j]], add=True)

        plsc.subcore_barrier()

        stripe = n // _NS
        pltpu.sync_copy(acc_sh.at[pl.ds(sid * stripe, stripe)],
                        out_hbm.at[cid, pl.ds(sid * stripe, stripe)])

    return deg_kernel(colp, wp, zinit)


def _sc_gather_scale_scatter(table, zinit, rowp, colp, wp):
    n, d = table.shape
    stripe = n // _NS
    mesh = plsc.VectorSubcoreMesh(core_axis_name="c", subcore_axis_name="s")
    cp = pltpu.CompilerParams()
    if "needs_layout_passes" in pltpu.CompilerParams.__dataclass_fields__:
        cp = dataclasses.replace(cp, needs_layout_passes=False)
    if "use_tc_tiling_on_sc" in pltpu.CompilerParams.__dataclass_fields__:
        cp = dataclasses.replace(cp, use_tc_tiling_on_sc=False)

    @functools.partial(
        pl.kernel,
        out_type=jax.ShapeDtypeStruct((_NC, n, d), jnp.float32),
        mesh=mesh,
        compiler_params=cp,
        scratch_types=[
            pltpu.VMEM((_KSUB, 128), jnp.int32),
            pltpu.VMEM((_KSUB, 128), jnp.int32),
            pltpu.VMEM((_KSUB, 128), jnp.float32),
            pltpu.VMEM((_CHUNK, d), jnp.float32),
            pltpu.SemaphoreType.DMA,
            pltpu.VMEM_SHARED((n, d), jnp.float32),
        ],
    )
    def seg_kernel(tab_hbm, z_hbm, row_hbm, col_hbm, w_hbm, out_hbm,
                   row_v, col_v, w_v, msg_v, gsem, acc_sh):
        cid = lax.axis_index("c")
        sid = lax.axis_index("s")
        wid = cid * _NS + sid

        @pl.when(cid == 0)
        def _():
            pltpu.sync_copy(tab_hbm.at[pl.ds(sid * stripe, stripe)],
                            acc_sh.at[pl.ds(sid * stripe, stripe)])

        @pl.when(cid != 0)
        def _():
            pltpu.sync_copy(z_hbm.at[pl.ds(sid * stripe, stripe)],
                            acc_sh.at[pl.ds(sid * stripe, stripe)])

        plsc.subcore_barrier()

        base_row = wid * _ROWS_PER_W

        @pl.loop(0, _G)
        def _(g):
            r0 = base_row + g * _KSUB
            pltpu.sync_copy(row_hbm.at[pl.ds(r0, _KSUB)], row_v)
            pltpu.sync_copy(col_hbm.at[pl.ds(r0, _KSUB)], col_v)
            pltpu.sync_copy(w_hbm.at[pl.ds(r0, _KSUB)], w_v)
            copies = []
            for j in range(_KSUB):
                copies.append(pltpu.async_copy(
                    tab_hbm.at[row_v.at[j]],
                    msg_v.at[pl.ds(j * 128, 128)], gsem))
            for c in copies:
                c.wait()

            @pl.loop(0, _KSUB)
            def _(j):
                jv = jnp.full((16,), j, jnp.int32)

                @pl.loop(0, 128)
                def _(i):
                    iv = jnp.full((16,), i, jnp.int32)
                    wv = plsc.load_gather(w_v, [jv, iv])
                    base = j * 128 + i
                    for t in range(d // 16):
                        sl = (base, pl.ds(t * 16, 16))
                        msg_v[sl] = msg_v[sl] * wv

            for j in range(_KSUB):
                pltpu.sync_copy(msg_v.at[pl.ds(j * 128, 128)],
                                acc_sh.at[col_v.at[j]], add=True)

        plsc.subcore_barrier()

        pltpu.sync_copy(acc_sh.at[pl.ds(sid * stripe, stripe)],
                        out_hbm.at[cid, pl.ds(sid * stripe, stripe)])

    return seg_kernel(table, zinit, rowp, colp, wp)


def _tc_matmul(x, w):
    n = x.shape[0]
    m = w.shape[1]

    def body(x_ref, w_ref, o_ref):
        o_ref[...] = jnp.dot(x_ref[...], w_ref[...],
                             preferred_element_type=jnp.float32)

    return pl.pallas_call(
        body, out_shape=jax.ShapeDtypeStruct((n, m), jnp.float32))(x, w)


def _tc_deg_scale(h, degt):
    n, m = h.shape

    def body(h_ref, d_ref, hp_ref, dis_ref):
        deg = d_ref[:, 0:1] + d_ref[:, 1:2] + 1.0
        dis = lax.rsqrt(deg)
        dis_ref[...] = dis
        hp_ref[...] = h_ref[...] * dis

    return pl.pallas_call(
        body,
        out_shape=[jax.ShapeDtypeStruct((n, m), jnp.float32),
                   jax.ShapeDtypeStruct((n, 1), jnp.float32)])(h, degt)


def _tc_layer1_finish(a0, a1, dis, b1, w2):
    n = a0.shape[0]
    m = w2.shape[1]

    def body(a0_ref, a1_ref, dis_ref, b_ref, w_ref, o_ref):
        z = jax.nn.relu((a0_ref[...] + a1_ref[...]) * dis_ref[...]
                        + b_ref[...])
        o_ref[...] = jnp.dot(z, w_ref[...],
                             preferred_element_type=jnp.float32) * dis_ref[...]

    return pl.pallas_call(
        body, out_shape=jax.ShapeDtypeStruct((n, m), jnp.float32))(
            a0, a1, dis, b1, w2)


def _tc_layer2_finish(a0, a1, dis, b2):
    n, m = a0.shape

    def body(a0_ref, a1_ref, dis_ref, b_ref, o_ref):
        o = (a0_ref[...] + a1_ref[...]) * dis_ref[...] + b_ref[...]
        mx = jnp.max(o, axis=1, keepdims=True)
        ex = jnp.exp(o - mx)
        lse = jnp.log(jnp.sum(ex, axis=1, keepdims=True)) + mx
        o_ref[...] = o - lse

    return pl.pallas_call(
        body, out_shape=jax.ShapeDtypeStruct((n, m), jnp.float32))(
            a0, a1, dis, b2)


def kernel(x, edge_index, edge_weight, W1, b1, W2, b2):
    n = x.shape[0]
    nhid = W1.shape[1]
    ncls = W2.shape[1]
    np_ = ((n + 16 * 128 - 1) // (16 * 128)) * (16 * 128)
    xp = jnp.pad(x, ((0, np_ - n), (0, 0)))

    rowp, colp, wp = _pad_edges(edge_index[0], edge_index[1], edge_weight, n)
    z1d = jnp.zeros((np_,), jnp.float32)
    z64 = jnp.zeros((np_, nhid), jnp.float32)
    z16 = jnp.zeros((np_, ncls), jnp.float32)

    degp = _sc_degree(colp, wp, z1d)
    h = _tc_matmul(xp, W1)
    h1p, dis = _tc_deg_scale(h, degp.T)
    acc1 = _sc_gather_scale_scatter(h1p, z64, rowp, colp, wp)
    h2p = _tc_layer1_finish(acc1[0], acc1[1], dis,
                            b1.reshape(1, nhid), W2)
    acc2 = _sc_gather_scale_scatter(h2p, z16, rowp, colp, wp)
    out = _tc_layer2_finish(acc2[0], acc2[1], dis,
                            b2.reshape(1, ncls))
    return out[:n]
